# baseline (speedup 1.0000x reference)
import functools
import math

import jax
import jax.numpy as jnp
from jax import lax
from jax.experimental import pallas as pl
from jax.experimental.pallas import tpu as pltpu

_INV_SQRT2 = 1.0 / math.sqrt(2.0)


def _gelu_exact(x):
    return 0.5 * x * (1.0 + lax.erf(x * _INV_SQRT2))


def _ffn_kernel(x_ref, w1_ref, b1_ref, w2_ref, b2_ref, o_ref, acc_ref):
    k = pl.program_id(1)

    @pl.when(k == 0)
    def _():
        acc_ref[...] = jnp.broadcast_to(b2_ref[...].astype(jnp.float32),
                                        acc_ref.shape)

    xb = x_ref[...].astype(jnp.bfloat16)
    w1c = w1_ref[...].astype(jnp.bfloat16)
    h = jnp.dot(xb, w1c, preferred_element_type=jnp.float32)
    h = _gelu_exact(h + b1_ref[...].astype(jnp.float32))
    w2c = w2_ref[...].astype(jnp.bfloat16)
    acc_ref[...] += jnp.dot(h.astype(jnp.bfloat16), w2c,
                            preferred_element_type=jnp.float32)

    @pl.when(k == pl.num_programs(1) - 1)
    def _():
        o_ref[...] = acc_ref[...].astype(o_ref.dtype)


def _single_buffered_spec_factory():
    try:
        pl.BlockSpec((8, 128), lambda i, k: (0, 0), pipeline_mode=pl.Buffered(1))

        def spec(shape, index_map):
            return pl.BlockSpec(shape, index_map, pipeline_mode=pl.Buffered(1))
        return spec
    except Exception:
        return pl.BlockSpec


def kernel(x, w1, b1, w2, b2):
    batch, seq, dim = x.shape
    hidden = w1.shape[1]
    M = batch * seq
    x2d = x.reshape(M, dim)

    b1r = b1.reshape(1, hidden).astype(jnp.float32)
    b2r = b2.reshape(1, dim).astype(jnp.float32)

    n_rows = 2
    TM = -(-M // n_rows)
    TM = -(-TM // 8) * 8
    Mp = n_rows * TM
    if Mp != M:
        x2d = jnp.pad(x2d, ((0, Mp - M), (0, 0)))

    th = 512 if hidden % 512 == 0 else hidden
    n_h = hidden // th

    cost = pl.CostEstimate(
        flops=int(4 * Mp * dim * hidden),
        transcendentals=int(Mp * hidden),
        bytes_accessed=int(4 * Mp * dim * 2 + 2 * (dim * hidden * 4)),
    )

    sspec = _single_buffered_spec_factory()

    out2d = pl.pallas_call(
        _ffn_kernel,
        out_shape=jax.ShapeDtypeStruct((Mp, dim), x.dtype),
        grid=(n_rows, n_h),
        in_specs=[
            sspec((TM, dim), lambda i, k: (i, 0)),
            pl.BlockSpec((dim, th), lambda i, k: (0, k)),
            pl.BlockSpec((1, th), lambda i, k: (0, k)),
            pl.BlockSpec((th, dim), lambda i, k: (k, 0)),
            pl.BlockSpec((1, dim), lambda i, k: (0, 0)),
        ],
        out_specs=sspec((TM, dim), lambda i, k: (i, 0)),
        scratch_shapes=[pltpu.VMEM((TM, dim), jnp.float32)],
        compiler_params=pltpu.CompilerParams(
            dimension_semantics=("parallel", "arbitrary"),
            vmem_limit_bytes=int(64 * 1024 * 1024 * 0.9),
        ),
        cost_estimate=cost,
    )(x2d, w1, b1r, w2, b2r)

    if Mp != M:
        out2d = out2d[:M]
    return out2d.reshape(batch, seq, dim)


# speedup vs baseline: 1.0329x; 1.0329x over previous
import functools
import math

import jax
import jax.numpy as jnp
from jax import lax
from jax.experimental import pallas as pl
from jax.experimental.pallas import tpu as pltpu

_INV_SQRT2 = 1.0 / math.sqrt(2.0)


def _gelu_exact(x):
    return 0.5 * x * (1.0 + lax.erf(x * _INV_SQRT2))


def _ffn_kernel(x_ref, w1_ref, b1_ref, w2_ref, b2_ref, o_ref, acc_ref, xb_ref,
                *, tm, nk):
    k = pl.program_id(1)
    j = pl.program_id(2)
    rows = pl.ds(j * tm, tm)

    @pl.when(k == 0)
    def _():
        xb_ref[rows, :] = x_ref[...].astype(jnp.bfloat16)
        acc_ref[rows, :] = jnp.broadcast_to(b2_ref[...].astype(jnp.float32),
                                            (tm, acc_ref.shape[1]))

    xb = xb_ref[rows, :]
    w1c = w1_ref[...].astype(jnp.bfloat16)
    h = jnp.dot(xb, w1c, preferred_element_type=jnp.float32)
    h = _gelu_exact(h + b1_ref[...].astype(jnp.float32))
    w2c = w2_ref[...].astype(jnp.bfloat16)
    acc = acc_ref[rows, :] + jnp.dot(h.astype(jnp.bfloat16), w2c,
                                     preferred_element_type=jnp.float32)
    acc_ref[rows, :] = acc

    @pl.when(k == nk - 1)
    def _():
        o_ref[...] = acc.astype(o_ref.dtype)


def kernel(x, w1, b1, w2, b2):
    batch, seq, dim = x.shape
    hidden = w1.shape[1]
    M = batch * seq
    x2d = x.reshape(M, dim)

    b1r = b1.reshape(1, hidden).astype(jnp.float32)
    b2r = b2.reshape(1, dim).astype(jnp.float32)

    nj = 4
    tm = 512
    n_blocks = -(-M // tm)
    if n_blocks % (2 * nj):
        n_blocks = -(-n_blocks // (2 * nj)) * (2 * nj)
    Mp = n_blocks * tm
    if Mp != M:
        x2d = jnp.pad(x2d, ((0, Mp - M), (0, 0)))

    th = 1024 if hidden % 1024 == 0 else hidden
    nk = hidden // th
    tm_core = (Mp // 2)

    cost = pl.CostEstimate(
        flops=int(4 * Mp * dim * hidden),
        transcendentals=int(Mp * hidden),
        bytes_accessed=int(4 * Mp * dim * 2 + 2 * (dim * hidden * 4)),
    )

    out2d = pl.pallas_call(
        functools.partial(_ffn_kernel, tm=tm, nk=nk),
        out_shape=jax.ShapeDtypeStruct((Mp, dim), x.dtype),
        grid=(2, nk, nj),
        in_specs=[
            pl.BlockSpec((tm, dim),
                         lambda i, k, j: (jnp.where(k == 0, i * 4 + j,
                                                    i * 4 + 3), 0)),
            pl.BlockSpec((dim, th), lambda i, k, j: (0, k)),
            pl.BlockSpec((1, th), lambda i, k, j: (0, k)),
            pl.BlockSpec((th, dim), lambda i, k, j: (k, 0)),
            pl.BlockSpec((1, dim), lambda i, k, j: (0, 0)),
        ],
        out_specs=pl.BlockSpec(
            (tm, dim),
            lambda i, k, j: (i * 4 + jnp.where(k == pl.num_programs(1) - 1,
                                               j, 0), 0)),
        scratch_shapes=[
            pltpu.VMEM((tm_core, dim), jnp.float32),
            pltpu.VMEM((tm_core, dim), jnp.bfloat16),
        ],
        compiler_params=pltpu.CompilerParams(
            dimension_semantics=("parallel", "arbitrary", "arbitrary"),
            vmem_limit_bytes=int(64 * 1024 * 1024 * 0.9),
        ),
        cost_estimate=cost,
    )(x2d, w1, b1r, w2, b2r)

    if Mp != M:
        out2d = out2d[:M]
    return out2d.reshape(batch, seq, dim)


# speedup vs baseline: 1.1233x; 1.0876x over previous
import functools
import math

import jax
import jax.numpy as jnp
from jax import lax
from jax.experimental import pallas as pl
from jax.experimental.pallas import tpu as pltpu

_INV_SQRT2 = 1.0 / math.sqrt(2.0)


def _gelu_exact(x):
    return 0.5 * x * (1.0 + lax.erf(x * _INV_SQRT2))


def _ffn_kernel(x_hbm, w1_hbm, b1_ref, w2_hbm, b2_ref, o_hbm,
                xin, xb, w1l, w2l, w1c, w2c, acc,
                sx, sw1, sw2, so, *, nk, nj, th, tmj, rows_core):
    i = pl.program_id(0)
    r0 = i * rows_core
    dim = acc.shape[1]

    def x_copy(j, slot):
        return pltpu.make_async_copy(
            x_hbm.at[pl.ds(r0 + j * tmj, tmj), :], xin.at[slot], sx.at[slot])

    def w1_copy(k, slot):
        return pltpu.make_async_copy(
            w1_hbm.at[:, pl.ds(k * th, th)], w1l.at[slot], sw1.at[slot])

    def w2_copy(k, slot):
        return pltpu.make_async_copy(
            w2_hbm.at[pl.ds(k * th, th), :], w2l.at[slot], sw2.at[slot])

    def o_copy(j):
        return pltpu.make_async_copy(
            acc.at[pl.ds(j * tmj, tmj), :],
            o_hbm.at[pl.ds(r0 + j * tmj, tmj), :], so.at[j])

    x_copy(0, 0).start()
    w1_copy(0, 0).start()
    w2_copy(0, 0).start()

    for k in range(nk):
        sl = k % 2
        w1_copy(k, sl).wait()
        w2_copy(k, sl).wait()
        w1c[...] = w1l[sl].astype(jnp.bfloat16)
        w2c[...] = w2l[sl].astype(jnp.bfloat16)
        if k + 1 < nk:
            w1_copy(k + 1, (k + 1) % 2).start()
            w2_copy(k + 1, (k + 1) % 2).start()
        b1k = b1_ref[:, pl.ds(k * th, th)].astype(jnp.float32)
        for j in range(nj):
            rows = pl.ds(j * tmj, tmj)
            if k == 0:
                x_copy(j, j % 2).wait()
                if j + 1 < nj:
                    x_copy(j + 1, (j + 1) % 2).start()
                xb[rows, :] = xin[j % 2].astype(jnp.bfloat16)
            h = jnp.dot(xb[rows, :], w1c[...],
                        preferred_element_type=jnp.float32)
            h = _gelu_exact(h + b1k)
            d = jnp.dot(h.astype(jnp.bfloat16), w2c[...],
                        preferred_element_type=jnp.float32)
            if k == 0:
                acc[rows, :] = d + b2_ref[...].astype(jnp.float32)
            else:
                acc[rows, :] += d
            if k == nk - 1:
                o_copy(j).start()

    for j in range(nj):
        o_copy(j).wait()


def kernel(x, w1, b1, w2, b2):
    batch, seq, dim = x.shape
    hidden = w1.shape[1]
    M = batch * seq
    x2d = x.reshape(M, dim)

    b1r = b1.reshape(1, hidden).astype(jnp.float32)
    b2r = b2.reshape(1, dim).astype(jnp.float32)

    nj = 4
    tmj = 512
    rows_core = nj * tmj
    Mp = -(-M // (2 * nj * tmj)) * (2 * nj * tmj)
    if Mp != M:
        x2d = jnp.pad(x2d, ((0, Mp - M), (0, 0)))
    rows_core = Mp // 2
    tmj = rows_core // nj

    th = 1024 if hidden % 1024 == 0 else hidden
    nk = hidden // th

    cost = pl.CostEstimate(
        flops=int(4 * Mp * dim * hidden),
        transcendentals=int(Mp * hidden),
        bytes_accessed=int(4 * Mp * dim * 2 + 2 * (dim * hidden * 4)),
    )

    out2d = pl.pallas_call(
        functools.partial(_ffn_kernel, nk=nk, nj=nj, th=th, tmj=tmj,
                          rows_core=rows_core),
        out_shape=jax.ShapeDtypeStruct((Mp, dim), x.dtype),
        grid=(2,),
        in_specs=[
            pl.BlockSpec(memory_space=pl.ANY),
            pl.BlockSpec(memory_space=pl.ANY),
            pl.BlockSpec((1, hidden), lambda i: (0, 0)),
            pl.BlockSpec(memory_space=pl.ANY),
            pl.BlockSpec((1, dim), lambda i: (0, 0)),
        ],
        out_specs=pl.BlockSpec(memory_space=pl.ANY),
        scratch_shapes=[
            pltpu.VMEM((2, tmj, dim), jnp.float32),
            pltpu.VMEM((rows_core, dim), jnp.bfloat16),
            pltpu.VMEM((2, dim, th), jnp.float32),
            pltpu.VMEM((2, th, dim), jnp.float32),
            pltpu.VMEM((dim, th), jnp.bfloat16),
            pltpu.VMEM((th, dim), jnp.bfloat16),
            pltpu.VMEM((rows_core, dim), jnp.float32),
            pltpu.SemaphoreType.DMA((2,)),
            pltpu.SemaphoreType.DMA((2,)),
            pltpu.SemaphoreType.DMA((2,)),
            pltpu.SemaphoreType.DMA((4,)),
        ],
        compiler_params=pltpu.CompilerParams(
            dimension_semantics=("parallel",),
            vmem_limit_bytes=int(64 * 1024 * 1024 * 0.9),
        ),
        cost_estimate=cost,
    )(x2d, w1, b1r, w2, b2r)

    if Mp != M:
        out2d = out2d[:M]
    return out2d.reshape(batch, seq, dim)
